# Initial kernel scaffold; baseline (speedup 1.0000x reference)
import jax
import jax.numpy as jnp
from jax.experimental import pallas as pl


def kernel(edge_index, feats_node, feats_graph, W1, b1, W2, b2, W3, b3, Wl1, bl1, Wl2, bl2, Wl3, bl3):
    raise NotImplementedError("write your pallas kernel here")


# speedup vs baseline: 2.9081x; 2.9081x over previous
import functools

import jax
import jax.numpy as jnp
from jax import lax
from jax.experimental import pallas as pl
from jax.experimental.pallas import tpu as pltpu
from jax.experimental.pallas import tpu_sc as plsc

N = 10000
E = 320000
D = 128
H = 128
EXTRA = 16

NC = 2
NS = 16
CH = 128
EPAD = NC * NS * 80 * CH
NROW = EPAD // CH
NPAD = 10240
RPT = NPAD // NS
DEG_ROWS = NROW // (NC * NS)
AGG_ROWS = NROW // (NC * NS)

_SELU_ALPHA = 1.6732632423543772
_SELU_SCALE = 1.0507009873554805

_MESH = dict(core_axis_name="c", subcore_axis_name="s", num_cores=NC,
             num_subcores=NS)


def _selu(x):
    return _SELU_SCALE * jnp.where(x > 0, x, _SELU_ALPHA * (jnp.exp(x) - 1.0))


@functools.partial(
    pl.kernel,
    out_type=[jax.ShapeDtypeStruct((NPAD,), jnp.float32),
              jax.ShapeDtypeStruct((NPAD,), jnp.float32),
              jax.ShapeDtypeStruct((NPAD,), jnp.float32),
              jax.ShapeDtypeStruct((NPAD,), jnp.float32)],
    mesh=plsc.VectorSubcoreMesh(**_MESH),
    scratch_types=[
        pltpu.VMEM((DEG_ROWS, CH), jnp.int32),
        pltpu.VMEM((DEG_ROWS, CH), jnp.int32),
        pltpu.VMEM((CH,), jnp.float32),
        pltpu.VMEM((RPT,), jnp.float32),
        pltpu.VMEM_SHARED((NPAD,), jnp.float32),
        pltpu.VMEM_SHARED((NPAD,), jnp.float32),
    ],
)
def _deg_kernel(src_h, dst_h, dgo0_h, dgo1_h, dgi0_h, dgi1_h,
                idxs_v, idxd_v, ones_v, zbuf_v, dego_sh, degi_sh):
    cid = lax.axis_index("c")
    sid = lax.axis_index("s")
    base = (cid * NS + sid) * DEG_ROWS

    for i in range(CH // 16):
        ones_v[pl.ds(i * 16, 16)] = jnp.ones((16,), jnp.float32)
    for i in range(RPT // 16):
        zbuf_v[pl.ds(i * 16, 16)] = jnp.zeros((16,), jnp.float32)
    pltpu.sync_copy(zbuf_v, dego_sh.at[pl.ds(sid * RPT, RPT)])
    pltpu.sync_copy(zbuf_v, degi_sh.at[pl.ds(sid * RPT, RPT)])
    pltpu.sync_copy(src_h.at[pl.ds(base, DEG_ROWS)], idxs_v)
    pltpu.sync_copy(dst_h.at[pl.ds(base, DEG_ROWS)], idxd_v)
    plsc.subcore_barrier()

    def body(j, carry):
        pltpu.sync_copy(ones_v, dego_sh.at[idxs_v.at[j]], add=True)
        pltpu.sync_copy(ones_v, degi_sh.at[idxd_v.at[j]], add=True)
        return carry

    lax.fori_loop(0, DEG_ROWS, body, 0)
    plsc.subcore_barrier()

    @pl.when(cid == 0)
    def _():
        pltpu.sync_copy(dego_sh.at[pl.ds(sid * RPT, RPT)],
                        dgo0_h.at[pl.ds(sid * RPT, RPT)])
        pltpu.sync_copy(degi_sh.at[pl.ds(sid * RPT, RPT)],
                        dgi0_h.at[pl.ds(sid * RPT, RPT)])

    @pl.when(cid == 1)
    def _():
        pltpu.sync_copy(dego_sh.at[pl.ds(sid * RPT, RPT)],
                        dgo1_h.at[pl.ds(sid * RPT, RPT)])
        pltpu.sync_copy(degi_sh.at[pl.ds(sid * RPT, RPT)],
                        dgi1_h.at[pl.ds(sid * RPT, RPT)])


@functools.partial(
    pl.kernel,
    out_type=[jax.ShapeDtypeStruct((NPAD, D), jnp.float32),
              jax.ShapeDtypeStruct((NPAD, D), jnp.float32)],
    mesh=plsc.VectorSubcoreMesh(**_MESH),
    scratch_types=[
        pltpu.VMEM((AGG_ROWS, CH), jnp.int32),
        pltpu.VMEM((AGG_ROWS, CH), jnp.int32),
        pltpu.VMEM((CH, D), jnp.float32),
        pltpu.VMEM_SHARED((NPAD, D), jnp.float32),
        pltpu.SemaphoreType.DMA,
    ],
)
def _agg_kernel(src_h, dst_h, xs_h, zeros_h, p0_h, p1_h, idxs_v, idxd_v,
                rows_v, agg_sh, gsem):
    cid = lax.axis_index("c")
    sid = lax.axis_index("s")
    base = (cid * NS + sid) * AGG_ROWS

    pltpu.sync_copy(zeros_h.at[pl.ds(sid * RPT, RPT)],
                    agg_sh.at[pl.ds(sid * RPT, RPT)])
    pltpu.sync_copy(src_h.at[pl.ds(base, AGG_ROWS)], idxs_v)
    pltpu.sync_copy(dst_h.at[pl.ds(base, AGG_ROWS)], idxd_v)
    plsc.subcore_barrier()

    def body(j, carry):
        pltpu.async_copy(xs_h.at[idxs_v.at[j]], rows_v, gsem).wait()
        pltpu.sync_copy(rows_v, agg_sh.at[idxd_v.at[j]], add=True)
        return carry

    lax.fori_loop(0, AGG_ROWS, body, 0)
    plsc.subcore_barrier()

    @pl.when(cid == 0)
    def _():
        pltpu.sync_copy(agg_sh.at[pl.ds(sid * RPT, RPT)],
                        p0_h.at[pl.ds(sid * RPT, RPT)])

    @pl.when(cid == 1)
    def _():
        pltpu.sync_copy(agg_sh.at[pl.ds(sid * RPT, RPT)],
                        p1_h.at[pl.ds(sid * RPT, RPT)])


def _prep_body(x_ref, dgo0_ref, dgo1_ref, dgi0_ref, dgi1_ref,
               xs_ref, so_ref, si_ref):
    dgo = dgo0_ref[...] + dgo1_ref[...]
    dgi = dgi0_ref[...] + dgi1_ref[...]
    so = lax.rsqrt(jnp.maximum(dgo, 1.0))
    si = lax.rsqrt(jnp.maximum(dgi, 1.0))
    so_ref[...] = so
    si_ref[...] = si
    xs_ref[...] = x_ref[...] * so


def _layer_body(p0_ref, p1_ref, si_ref, so_ref, w_ref, b_ref, out_ref):
    agg = (p0_ref[...] + p1_ref[...]) * si_ref[...]
    z = jnp.dot(agg, w_ref[...], preferred_element_type=jnp.float32)
    out_ref[...] = _selu(z + b_ref[...]) * so_ref[...]


def _final_body(p0_ref, p1_ref, si_ref, w3_ref, b3_ref, fg_ref, wl1_ref,
                bl1_ref, wl2_ref, bl2_ref, wl3_ref, bl3_ref, out_ref):
    agg = (p0_ref[pl.ds(0, N), :] + p1_ref[pl.ds(0, N), :]) \
        * si_ref[pl.ds(0, N), :]
    h = jnp.dot(agg, w3_ref[...], preferred_element_type=jnp.float32)
    h = h + b3_ref[...]
    emb = jnp.mean(h, axis=0, keepdims=True)
    t = (jnp.dot(emb, wl1_ref[pl.ds(0, H), :],
                 preferred_element_type=jnp.float32)
         + jnp.dot(fg_ref[...], wl1_ref[pl.ds(H, EXTRA), :],
                   preferred_element_type=jnp.float32)
         + bl1_ref[...])
    t = _selu(t)
    t = _selu(jnp.dot(t, wl2_ref[...], preferred_element_type=jnp.float32)
              + bl2_ref[...])
    out_ref[...] = (jnp.dot(t, wl3_ref[...],
                            preferred_element_type=jnp.float32)
                    + bl3_ref[...])


def kernel(edge_index, feats_node, feats_graph, W1, b1, W2, b2, W3, b3,
           Wl1, bl1, Wl2, bl2, Wl3, bl3):
    f32 = jnp.float32
    pad_idx = jnp.full((EPAD - E,), NPAD - 1, jnp.int32)
    src = jnp.concatenate([edge_index[0], pad_idx]).reshape(NROW, CH)
    dst = jnp.concatenate([edge_index[1], pad_idx]).reshape(NROW, CH)
    x_pad = jnp.concatenate(
        [feats_node, jnp.zeros((NPAD - N, D), f32)], axis=0)
    zeros2d = jnp.zeros((NPAD, D), f32)

    dgo0, dgo1, dgi0, dgi1 = _deg_kernel(src, dst)

    xs0, so, si = pl.pallas_call(
        _prep_body,
        out_shape=[jax.ShapeDtypeStruct((NPAD, D), f32),
                   jax.ShapeDtypeStruct((NPAD, 1), f32),
                   jax.ShapeDtypeStruct((NPAD, 1), f32)],
    )(x_pad, dgo0.reshape(NPAD, 1), dgo1.reshape(NPAD, 1),
      dgi0.reshape(NPAD, 1), dgi1.reshape(NPAD, 1))

    layer = pl.pallas_call(
        _layer_body,
        out_shape=jax.ShapeDtypeStruct((NPAD, D), f32),
    )

    x = xs0
    for W, b in ((W1, b1), (W2, b2)):
        pa, pb = _agg_kernel(src, dst, x, zeros2d)
        x = layer(pa, pb, si, so, W, b.reshape(1, H))

    pa, pb = _agg_kernel(src, dst, x, zeros2d)
    out = pl.pallas_call(
        _final_body,
        out_shape=jax.ShapeDtypeStruct((1, 1), f32),
    )(pa, pb, si, W3, b3.reshape(1, H), feats_graph, Wl1,
      bl1.reshape(1, 2 * H), Wl2, bl2.reshape(1, H), Wl3, bl3.reshape(1, 1))
    return out.reshape(-1)


# speedup vs baseline: 11.2818x; 3.8795x over previous
import functools

import jax
import jax.numpy as jnp
from jax import lax
from jax.experimental import pallas as pl
from jax.experimental.pallas import tpu as pltpu
from jax.experimental.pallas import tpu_sc as plsc

N = 10000
E = 320000
D = 128
H = 128
EXTRA = 16

NC = 2
NS = 16
CH = 128
EPAD = NC * NS * 80 * CH
NROW = EPAD // CH
NPAD = 10240
RPT = NPAD // NS
DEG_ROWS = NROW // (NC * NS)
AGG_ROWS = NROW // (NC * NS)

_SELU_ALPHA = 1.6732632423543772
_SELU_SCALE = 1.0507009873554805

_MESH = dict(core_axis_name="c", subcore_axis_name="s", num_cores=NC,
             num_subcores=NS)


def _selu(x):
    return _SELU_SCALE * jnp.where(x > 0, x, _SELU_ALPHA * (jnp.exp(x) - 1.0))


@functools.partial(
    pl.kernel,
    out_type=[jax.ShapeDtypeStruct((NPAD,), jnp.float32),
              jax.ShapeDtypeStruct((NPAD,), jnp.float32),
              jax.ShapeDtypeStruct((NPAD,), jnp.float32),
              jax.ShapeDtypeStruct((NPAD,), jnp.float32)],
    mesh=plsc.VectorSubcoreMesh(**_MESH),
    scratch_types=[
        pltpu.VMEM((DEG_ROWS, CH), jnp.int32),
        pltpu.VMEM((DEG_ROWS, CH), jnp.int32),
        pltpu.VMEM((CH,), jnp.float32),
        pltpu.VMEM((RPT,), jnp.float32),
        pltpu.VMEM_SHARED((NPAD,), jnp.float32),
        pltpu.VMEM_SHARED((NPAD,), jnp.float32),
    ],
)
def _deg_kernel(src_h, dst_h, dgo0_h, dgo1_h, dgi0_h, dgi1_h,
                idxs_v, idxd_v, ones_v, zbuf_v, dego_sh, degi_sh):
    cid = lax.axis_index("c")
    sid = lax.axis_index("s")
    base = (cid * NS + sid) * DEG_ROWS

    for i in range(CH // 16):
        ones_v[pl.ds(i * 16, 16)] = jnp.ones((16,), jnp.float32)
    for i in range(RPT // 16):
        zbuf_v[pl.ds(i * 16, 16)] = jnp.zeros((16,), jnp.float32)
    pltpu.sync_copy(zbuf_v, dego_sh.at[pl.ds(sid * RPT, RPT)])
    pltpu.sync_copy(zbuf_v, degi_sh.at[pl.ds(sid * RPT, RPT)])
    pltpu.sync_copy(src_h.at[pl.ds(base, DEG_ROWS)], idxs_v)
    pltpu.sync_copy(dst_h.at[pl.ds(base, DEG_ROWS)], idxd_v)
    plsc.subcore_barrier()

    def body(j, carry):
        pltpu.sync_copy(ones_v, dego_sh.at[idxs_v.at[j]], add=True)
        pltpu.sync_copy(ones_v, degi_sh.at[idxd_v.at[j]], add=True)
        return carry

    lax.fori_loop(0, DEG_ROWS, body, 0)
    plsc.subcore_barrier()

    @pl.when(cid == 0)
    def _():
        pltpu.sync_copy(dego_sh.at[pl.ds(sid * RPT, RPT)],
                        dgo0_h.at[pl.ds(sid * RPT, RPT)])
        pltpu.sync_copy(degi_sh.at[pl.ds(sid * RPT, RPT)],
                        dgi0_h.at[pl.ds(sid * RPT, RPT)])

    @pl.when(cid == 1)
    def _():
        pltpu.sync_copy(dego_sh.at[pl.ds(sid * RPT, RPT)],
                        dgo1_h.at[pl.ds(sid * RPT, RPT)])
        pltpu.sync_copy(degi_sh.at[pl.ds(sid * RPT, RPT)],
                        dgi1_h.at[pl.ds(sid * RPT, RPT)])


@functools.partial(
    pl.kernel,
    out_type=[jax.ShapeDtypeStruct((NPAD, D), jnp.float32),
              jax.ShapeDtypeStruct((NPAD, D), jnp.float32)],
    mesh=plsc.VectorSubcoreMesh(**_MESH),
    scratch_types=[
        pltpu.VMEM((AGG_ROWS // 2, CH), jnp.int32),
        pltpu.VMEM((AGG_ROWS // 2, CH), jnp.int32),
        pltpu.VMEM((CH, D), jnp.float32),
        pltpu.VMEM((CH, D), jnp.float32),
        pltpu.VMEM_SHARED((NPAD, D), jnp.float32),
        pltpu.SemaphoreType.DMA,
        pltpu.SemaphoreType.DMA,
    ],
)
def _agg_kernel(src_h, dst_h, xs_h, zeros_h, p0_h, p1_h, idxs_v, idxd_v,
                rows0_v, rows1_v, agg_sh, gsem0, gsem1):
    cid = lax.axis_index("c")
    sid = lax.axis_index("s")
    base = (cid * NS + sid) * AGG_ROWS
    PH = AGG_ROWS // 2

    pltpu.sync_copy(zeros_h.at[pl.ds(sid * RPT, RPT)],
                    agg_sh.at[pl.ds(sid * RPT, RPT)])

    for phase in range(2):
        pltpu.sync_copy(src_h.at[pl.ds(base + phase * PH, PH)], idxs_v)
        pltpu.sync_copy(dst_h.at[pl.ds(base + phase * PH, PH)], idxd_v)
        if phase == 0:
            plsc.subcore_barrier()
        pltpu.async_copy(xs_h.at[idxs_v.at[0]], rows0_v, gsem0)

        def body(i, carry):
            j = i * 2
            pltpu.async_copy(xs_h.at[idxs_v.at[j + 1]], rows1_v, gsem1)
            pltpu.make_async_copy(xs_h.at[idxs_v.at[j]], rows0_v,
                                  gsem0).wait()
            pltpu.sync_copy(rows0_v, agg_sh.at[idxd_v.at[j]], add=True)

            @pl.when(j + 2 < PH)
            def _():
                pltpu.async_copy(xs_h.at[idxs_v.at[j + 2]], rows0_v, gsem0)

            pltpu.make_async_copy(xs_h.at[idxs_v.at[j + 1]], rows1_v,
                                  gsem1).wait()
            pltpu.sync_copy(rows1_v, agg_sh.at[idxd_v.at[j + 1]], add=True)
            return carry

        lax.fori_loop(0, PH // 2, body, 0)
    plsc.subcore_barrier()

    @pl.when(cid == 0)
    def _():
        pltpu.sync_copy(agg_sh.at[pl.ds(sid * RPT, RPT)],
                        p0_h.at[pl.ds(sid * RPT, RPT)])

    @pl.when(cid == 1)
    def _():
        pltpu.sync_copy(agg_sh.at[pl.ds(sid * RPT, RPT)],
                        p1_h.at[pl.ds(sid * RPT, RPT)])


def _prep_body(x_ref, dgo0_ref, dgo1_ref, dgi0_ref, dgi1_ref,
               xs_ref, so_ref, si_ref):
    dgo = dgo0_ref[...] + dgo1_ref[...]
    dgi = dgi0_ref[...] + dgi1_ref[...]
    so = lax.rsqrt(jnp.maximum(dgo, 1.0))
    si = lax.rsqrt(jnp.maximum(dgi, 1.0))
    so_ref[...] = so
    si_ref[...] = si
    xs_ref[...] = x_ref[...] * so


def _layer_body(p0_ref, p1_ref, si_ref, so_ref, w_ref, b_ref, out_ref):
    agg = (p0_ref[...] + p1_ref[...]) * si_ref[...]
    z = jnp.dot(agg, w_ref[...], preferred_element_type=jnp.float32)
    out_ref[...] = _selu(z + b_ref[...]) * so_ref[...]


def _final_body(p0_ref, p1_ref, si_ref, w3_ref, b3_ref, fg_ref, wl1_ref,
                bl1_ref, wl2_ref, bl2_ref, wl3_ref, bl3_ref, out_ref):
    agg = (p0_ref[pl.ds(0, N), :] + p1_ref[pl.ds(0, N), :]) \
        * si_ref[pl.ds(0, N), :]
    h = jnp.dot(agg, w3_ref[...], preferred_element_type=jnp.float32)
    h = h + b3_ref[...]
    emb = jnp.mean(h, axis=0, keepdims=True)
    t = (jnp.dot(emb, wl1_ref[pl.ds(0, H), :],
                 preferred_element_type=jnp.float32)
         + jnp.dot(fg_ref[...], wl1_ref[pl.ds(H, EXTRA), :],
                   preferred_element_type=jnp.float32)
         + bl1_ref[...])
    t = _selu(t)
    t = _selu(jnp.dot(t, wl2_ref[...], preferred_element_type=jnp.float32)
              + bl2_ref[...])
    out_ref[...] = (jnp.dot(t, wl3_ref[...],
                            preferred_element_type=jnp.float32)
                    + bl3_ref[...])


def kernel(edge_index, feats_node, feats_graph, W1, b1, W2, b2, W3, b3,
           Wl1, bl1, Wl2, bl2, Wl3, bl3):
    f32 = jnp.float32
    pad_idx = N + jnp.arange(EPAD - E, dtype=jnp.int32) % (NPAD - N)
    src = jnp.concatenate([edge_index[0], pad_idx]).reshape(NROW, CH)
    dst = jnp.concatenate([edge_index[1], pad_idx]).reshape(NROW, CH)
    x_pad = jnp.concatenate(
        [feats_node, jnp.zeros((NPAD - N, D), f32)], axis=0)
    zeros2d = jnp.zeros((NPAD, D), f32)

    dgo0, dgo1, dgi0, dgi1 = _deg_kernel(src, dst)

    xs0, so, si = pl.pallas_call(
        _prep_body,
        out_shape=[jax.ShapeDtypeStruct((NPAD, D), f32),
                   jax.ShapeDtypeStruct((NPAD, 1), f32),
                   jax.ShapeDtypeStruct((NPAD, 1), f32)],
    )(x_pad, dgo0.reshape(NPAD, 1), dgo1.reshape(NPAD, 1),
      dgi0.reshape(NPAD, 1), dgi1.reshape(NPAD, 1))

    layer = pl.pallas_call(
        _layer_body,
        out_shape=jax.ShapeDtypeStruct((NPAD, D), f32),
    )

    x = xs0
    for W, b in ((W1, b1), (W2, b2)):
        pa, pb = _agg_kernel(src, dst, x, zeros2d)
        x = layer(pa, pb, si, so, W, b.reshape(1, H))

    pa, pb = _agg_kernel(src, dst, x, zeros2d)
    out = pl.pallas_call(
        _final_body,
        out_shape=jax.ShapeDtypeStruct((1, 1), f32),
    )(pa, pb, si, W3, b3.reshape(1, H), feats_graph, Wl1,
      bl1.reshape(1, 2 * H), Wl2, bl2.reshape(1, H), Wl3, bl3.reshape(1, 1))
    return out.reshape(-1)
